# baseline (speedup 1.0000x reference)
import functools

import jax
import jax.numpy as jnp
import numpy as np
from jax.experimental import pallas as pl
from jax.experimental.pallas import tpu as pltpu

_B = 128
_D = 100000
_SUB = 8
_W = _D // _SUB
_CH = 1664
_NCH = 8
_WP = _NCH * _CH
_PAD = _WP - _W
_LAST = (_NCH - 1) * _CH
_NEG = -1.0e30
_NEWTON_ITERS = 10


@functools.cache
def _gumbels():
    with jax.default_device(jax.devices("cpu")[0]), \
         jax.ensure_compile_time_eval():
        e = jax.random.exponential(
            jax.random.key(42), (_B, _D), dtype=jnp.float32
        )
        g = (-jnp.log(e) * 0.01).reshape(_B, _SUB, _W)
        return np.asarray(g)


def _tree(acc, final):
    w = acc.shape[-1]
    while w > 128:
        h = ((w // 128 + 1) // 2) * 128
        lo = acc[:, :h]
        hi = acc[:, h:]
        if w - h < h:
            fill = _NEG if final is jnp.max else 0.0
            hi = jnp.concatenate(
                [hi, jnp.full((_SUB, 2 * h - w), fill, jnp.float32)], axis=-1
            )
        acc = jnp.maximum(lo, hi) if final is jnp.max else lo + hi
        w = h
    return final(acc)


def _padded_chunk(x, k):
    if k < _NCH - 1:
        return x[:, k * _CH:(k + 1) * _CH]
    c = x[:, _LAST:_W]
    return jnp.concatenate(
        [c, jnp.full((_SUB, _PAD), _NEG, x.dtype)], axis=-1
    )


def _row_body(s_ref, n_ref, sample_ref, ent_ref, gp_ref):
    s = s_ref[0]
    n = n_ref[0]

    ms_acc = None
    mg_acc = None
    for k in range(_NCH):
        sc = _padded_chunk(s, k)
        gc = sc + _padded_chunk(n, k)
        gp_ref[:, k * _CH:(k + 1) * _CH] = gc
        ms_acc = sc if ms_acc is None else jnp.maximum(ms_acc, sc)
        mg_acc = gc if mg_acc is None else jnp.maximum(mg_acc, gc)
    ms = _tree(ms_acc, jnp.max)
    big_m = _tree(mg_acc, jnp.max)

    e_acc = jnp.zeros((_SUB, _CH), jnp.float32)
    e1_acc = jnp.zeros((_SUB, _CH), jnp.float32)
    for k in range(_NCH):
        cm = _padded_chunk(s, k) - ms
        e = jnp.exp(cm)
        e_acc = e_acc + e
        e1_acc = e1_acc + cm * e
    s0 = _tree(e_acc, jnp.sum)
    s1 = _tree(e1_acc, jnp.sum)
    ent = jnp.log(s0) - s1 / s0
    ent_ref[0] = jnp.full((1, 128), ent, dtype=jnp.float32)

    t = big_m - 1.0
    for _ in range(_NEWTON_ITERS):
        a_acc = jnp.zeros((_SUB, _CH), jnp.float32)
        n_acc = jnp.zeros((_SUB, _CH), jnp.float32)
        for k in range(_NCH):
            c = gp_ref[:, k * _CH:(k + 1) * _CH]
            a_acc = a_acc + jnp.maximum(c - t, 0.0)
            n_acc = n_acc + jnp.where(c > t, 1.0, 0.0)
        a_sum = _tree(a_acc, jnp.sum)
        n_sum = _tree(n_acc, jnp.sum)
        t = jnp.where(n_sum > 0.0, t + (a_sum - 1.0) / n_sum, t)

    for k in range(_NCH - 1):
        sample_ref[0, :, k * _CH:(k + 1) * _CH] = jnp.maximum(
            gp_ref[:, k * _CH:(k + 1) * _CH] - t, 0.0
        )
    sample_ref[0, :, _LAST:_W] = jnp.maximum(
        gp_ref[:, _LAST:_W] - t, 0.0
    )


def kernel(scores):
    s3 = scores.reshape(_B, _SUB, _W)
    sample3, ent3 = pl.pallas_call(
        _row_body,
        grid=(_B,),
        in_specs=[
            pl.BlockSpec((1, _SUB, _W), lambda i: (i, 0, 0)),
            pl.BlockSpec((1, _SUB, _W), lambda i: (i, 0, 0)),
        ],
        out_specs=[
            pl.BlockSpec((1, _SUB, _W), lambda i: (i, 0, 0)),
            pl.BlockSpec((1, 1, 128), lambda i: (i, 0, 0)),
        ],
        out_shape=[
            jax.ShapeDtypeStruct((_B, _SUB, _W), jnp.float32),
            jax.ShapeDtypeStruct((_B, 1, 128), jnp.float32),
        ],
        scratch_shapes=[pltpu.VMEM((_SUB, _WP), jnp.float32)],
    )(s3, _gumbels())
    sample = sample3.reshape(_B, _D)
    entropy = ent3[:, 0, 0]
    return (sample, scores, entropy)


# speedup vs baseline: 1.0248x; 1.0248x over previous
import functools

import jax
import jax.numpy as jnp
import numpy as np
from jax.experimental import pallas as pl
from jax.experimental.pallas import tpu as pltpu

_B = 128
_D = 100000
_SUB = 8
_W = _D // _SUB
_CH = 1664
_NCH = 8
_WP = _NCH * _CH
_PAD = _WP - _W
_LAST = (_NCH - 1) * _CH
_NEG = -1.0e30
_NEWTON_ITERS = 10
_RPS = 2


@functools.cache
def _gumbels():
    with jax.default_device(jax.devices("cpu")[0]), \
         jax.ensure_compile_time_eval():
        e = jax.random.exponential(
            jax.random.key(42), (_B, _D), dtype=jnp.float32
        )
        g = (-jnp.log(e) * 0.01).reshape(_B, _SUB, _W)
        return np.asarray(g)


def _tree(acc, final):
    w = acc.shape[-1]
    while w > 128:
        h = ((w // 128 + 1) // 2) * 128
        lo = acc[:, :h]
        hi = acc[:, h:]
        if w - h < h:
            fill = _NEG if final is jnp.max else 0.0
            hi = jnp.concatenate(
                [hi, jnp.full((_SUB, 2 * h - w), fill, jnp.float32)], axis=-1
            )
        acc = jnp.maximum(lo, hi) if final is jnp.max else lo + hi
        w = h
    return final(acc)


def _padded_chunk(x, k):
    if k < _NCH - 1:
        return x[:, k * _CH:(k + 1) * _CH]
    c = x[:, _LAST:_W]
    return jnp.concatenate(
        [c, jnp.full((_SUB, _PAD), _NEG, x.dtype)], axis=-1
    )


def _one_row(s, n, r, sample_ref, ent_ref, gp_ref):
    ms_acc = None
    mg_acc = None
    for k in range(_NCH):
        sc = _padded_chunk(s, k)
        gc = sc + _padded_chunk(n, k)
        gp_ref[r, :, k * _CH:(k + 1) * _CH] = gc
        ms_acc = sc if ms_acc is None else jnp.maximum(ms_acc, sc)
        mg_acc = gc if mg_acc is None else jnp.maximum(mg_acc, gc)
    ms = _tree(ms_acc, jnp.max)
    big_m = _tree(mg_acc, jnp.max)

    e_acc = jnp.zeros((_SUB, _CH), jnp.float32)
    e1_acc = jnp.zeros((_SUB, _CH), jnp.float32)
    for k in range(_NCH):
        cm = _padded_chunk(s, k) - ms
        e = jnp.exp(cm)
        e_acc = e_acc + e
        e1_acc = e1_acc + cm * e
    s0 = _tree(e_acc, jnp.sum)
    s1 = _tree(e1_acc, jnp.sum)
    ent = jnp.log(s0) - s1 / s0
    ent_ref[r] = jnp.full((1, 128), ent, dtype=jnp.float32)

    t = big_m - 1.0
    for _ in range(_NEWTON_ITERS):
        a_acc = jnp.zeros((_SUB, _CH), jnp.float32)
        n_acc = jnp.zeros((_SUB, _CH), jnp.float32)
        for k in range(_NCH):
            c = gp_ref[r, :, k * _CH:(k + 1) * _CH]
            a_acc = a_acc + jnp.maximum(c - t, 0.0)
            n_acc = n_acc + jnp.where(c > t, 1.0, 0.0)
        a_sum = _tree(a_acc, jnp.sum)
        n_sum = _tree(n_acc, jnp.sum)
        t = jnp.where(n_sum > 0.0, t + (a_sum - 1.0) / n_sum, t)

    for k in range(_NCH - 1):
        sample_ref[r, :, k * _CH:(k + 1) * _CH] = jnp.maximum(
            gp_ref[r, :, k * _CH:(k + 1) * _CH] - t, 0.0
        )
    sample_ref[r, :, _LAST:_W] = jnp.maximum(
        gp_ref[r, :, _LAST:_W] - t, 0.0
    )


def _row_body(s_ref, n_ref, sample_ref, ent_ref, gp_ref):
    for r in range(_RPS):
        _one_row(s_ref[r], n_ref[r], r, sample_ref, ent_ref, gp_ref)


def kernel(scores):
    s3 = scores.reshape(_B, _SUB, _W)
    sample3, ent3 = pl.pallas_call(
        _row_body,
        grid=(_B // _RPS,),
        in_specs=[
            pl.BlockSpec((_RPS, _SUB, _W), lambda i: (i, 0, 0)),
            pl.BlockSpec((_RPS, _SUB, _W), lambda i: (i, 0, 0)),
        ],
        out_specs=[
            pl.BlockSpec((_RPS, _SUB, _W), lambda i: (i, 0, 0)),
            pl.BlockSpec((_RPS, 1, 128), lambda i: (i, 0, 0)),
        ],
        out_shape=[
            jax.ShapeDtypeStruct((_B, _SUB, _W), jnp.float32),
            jax.ShapeDtypeStruct((_B, 1, 128), jnp.float32),
        ],
        scratch_shapes=[pltpu.VMEM((_RPS, _SUB, _WP), jnp.float32)],
    )(s3, _gumbels())
    sample = sample3.reshape(_B, _D)
    entropy = ent3[:, 0, 0]
    return (sample, scores, entropy)
